# baseline (speedup 1.0000x reference)
import functools

import jax
import jax.numpy as jnp
from jax import lax
from jax.experimental import pallas as pl
from jax.experimental.pallas import tpu as pltpu
from jax.experimental.pallas import tpu_sc as plsc

N = 10000
E = 320000
D = 128
NC, NS = 2, 16
NW = NC * NS
CH = 80
NPAD = 10240
RPT = NPAD // NS
ZR = 64
BM = 2000
BN = 2000
HI = lax.Precision.HIGHEST

SPLITS = (192000, 128000)

_MESH = plsc.VectorSubcoreMesh(core_axis_name="c", subcore_axis_name="s")


def _make_sc_edge(call_base, ecall):
    epw = ecall // NW
    nchunk = epw // CH

    @functools.partial(
        pl.kernel,
        mesh=_MESH,
        out_type=jax.ShapeDtypeStruct((ecall, D), jnp.float32),
        scratch_types=(
            [pltpu.VMEM((epw,), jnp.int32)] * 2
            + [pltpu.VMEM((CH, D), jnp.float32)] * 9
            + [pltpu.SemaphoreType.DMA] * 9
        ),
    )
    def sc_edge(p_hbm, q_hbm, recv_hbm, send_hbm, h1_hbm,
                ridx, sidx, prow0, prow1, prow2, qrow0, qrow1, qrow2,
                h1v0, h1v1, h1v2,
                gp0, gp1, gp2, gq0, gq1, gq2, wr0, wr1, wr2):
        wid = lax.axis_index("s") * NC + lax.axis_index("c")
        gbase = call_base + wid * epw
        obase = wid * epw
        prow = (prow0, prow1, prow2)
        qrow = (qrow0, qrow1, qrow2)
        h1v = (h1v0, h1v1, h1v2)
        gp = (gp0, gp1, gp2)
        gq = (gq0, gq1, gq2)
        wr = (wr0, wr1, wr2)

        pltpu.sync_copy(recv_hbm.at[pl.ds(gbase, epw)], ridx)
        pltpu.sync_copy(send_hbm.at[pl.ds(gbase, epw)], sidx)

        def issue_gather(cc, b):
            off = cc * CH
            pltpu.async_copy(p_hbm.at[ridx.at[pl.ds(off, CH)]], prow[b], gp[b])
            pltpu.async_copy(q_hbm.at[sidx.at[pl.ds(off, CH)]], qrow[b], gq[b])

        def wait_gather(b):
            pltpu.make_async_copy(p_hbm.at[pl.ds(0, CH)], prow[b], gp[b]).wait()
            pltpu.make_async_copy(q_hbm.at[pl.ds(0, CH)], qrow[b], gq[b]).wait()

        def issue_wb(cc, b):
            pltpu.async_copy(h1v[b], h1_hbm.at[pl.ds(obase + cc * CH, CH)], wr[b])

        def wait_wb(b):
            pltpu.make_async_copy(h1v[b], h1_hbm.at[pl.ds(obase, CH)], wr[b]).wait()

        issue_gather(0, 0)
        issue_gather(1, 1)

        @pl.loop(0, nchunk + 2, step=3)
        def _(k):
            for j in range(3):
                b = j
                cc = k + j

                @pl.when(cc < nchunk)
                def _():
                    @pl.when(cc + 2 < nchunk)
                    def _():
                        issue_gather(cc + 2, (b + 2) % 3)

                    wait_gather(b)

                    @pl.when(cc >= 3)
                    def _():
                        wait_wb(b)

                    @pl.loop(0, CH, step=2)
                    def _(r):
                        for rr in range(2):
                            for c in range(0, D, 16):
                                a = prow[b].at[pl.ds(r + rr, 1), pl.ds(c, 16)][...]
                                q = qrow[b].at[pl.ds(r + rr, 1), pl.ds(c, 16)][...]
                                h1v[b].at[pl.ds(r + rr, 1), pl.ds(c, 16)][...] = (
                                    jnp.maximum(a + q, 0.0))

                    issue_wb(cc, b)

        wait_wb((nchunk - 3) % 3)
        wait_wb((nchunk - 2) % 3)
        wait_wb((nchunk - 1) % 3)

    return sc_edge


def _make_sc_scatter(call_base, ecall):
    epw = ecall // NW
    nchunk = epw // CH

    @functools.partial(
        pl.kernel,
        mesh=_MESH,
        out_type=jax.ShapeDtypeStruct((NC, NPAD, D), jnp.float32),
        scratch_types=(
            [pltpu.VMEM_SHARED((NPAD, D), jnp.float32),
             pltpu.VMEM((epw,), jnp.int32)]
            + [pltpu.VMEM((CH, D), jnp.float32)] * 3
            + [pltpu.VMEM((ZR, D), jnp.float32)]
            + [pltpu.SemaphoreType.DMA] * 6
        ),
    )
    def sc_scatter(h2_hbm, recv_hbm, out_hbm, acc_sh, cidx, upd0, upd1, upd2,
                   stage, ld0, ld1, ld2, sc0, sc1, sc2):
        cid = lax.axis_index("c")
        sid = lax.axis_index("s")
        upd = (upd0, upd1, upd2)
        ld = (ld0, ld1, ld2)
        sc = (sc0, sc1, sc2)

        wid = sid * NC + cid
        gbase = call_base + wid * epw
        hbase = wid * epw

        @pl.loop(0, ZR)
        def _(r):
            for c in range(0, D, 16):
                stage.at[pl.ds(r, 1), pl.ds(c, 16)][...] = (
                    jnp.zeros((1, 16), jnp.float32))

        rbase = sid * RPT

        @pl.loop(0, RPT, step=ZR)
        def _(r):
            pltpu.sync_copy(stage, acc_sh.at[pl.ds(rbase + r, ZR)])

        pltpu.sync_copy(recv_hbm.at[pl.ds(gbase, epw)], cidx)
        plsc.subcore_barrier()

        def issue_load(cc, b):
            pltpu.async_copy(h2_hbm.at[pl.ds(hbase + cc * CH, CH)], upd[b], ld[b])

        def wait_load(b):
            pltpu.make_async_copy(h2_hbm.at[pl.ds(hbase, CH)], upd[b], ld[b]).wait()

        def issue_scatter(cc, b):
            off = cc * CH
            pltpu.async_copy(upd[b], acc_sh.at[cidx.at[pl.ds(off, CH)]], sc[b],
                             add=True)

        def wait_scatter(b):
            pltpu.make_async_copy(h2_hbm.at[pl.ds(hbase, CH)], upd[b], sc[b]).wait()

        issue_load(0, 0)

        @pl.loop(0, nchunk + 2, step=3)
        def _(k):
            for j in range(3):
                b = j
                cc = k + j

                @pl.when(cc < nchunk)
                def _():
                    wait_load(b)
                    issue_scatter(cc, b)

                    @pl.when(cc + 1 < nchunk)
                    def _():
                        @pl.when(cc >= 2)
                        def _():
                            wait_scatter((b + 1) % 3)

                        issue_load(cc + 1, (b + 1) % 3)

        wait_scatter((nchunk - 3) % 3)
        wait_scatter((nchunk - 2) % 3)
        wait_scatter((nchunk - 1) % 3)
        plsc.subcore_barrier()

        @pl.loop(0, RPT, step=ZR)
        def _(r):
            pltpu.sync_copy(acc_sh.at[pl.ds(rbase + r, ZR)], stage)
            pltpu.sync_copy(stage, out_hbm.at[cid, pl.ds(rbase + r, ZR)])

    return sc_scatter


_EDGE_CALLS = []
_SCATTER_CALLS = []
_base = 0
for _ec in SPLITS:
    _EDGE_CALLS.append(_make_sc_edge(_base, _ec))
    _SCATTER_CALLS.append(_make_sc_scatter(_base, _ec))
    _base += _ec


def _pre_body(x_ref, w1r_ref, w1s_ref, b1_ref, p_ref, q_ref):
    x = x_ref[...]
    p_ref[...] = jnp.dot(x, w1r_ref[...], precision=HI) + b1_ref[...]
    q_ref[...] = jnp.dot(x, w1s_ref[...], precision=HI)


_pre = pl.pallas_call(
    _pre_body,
    grid=(N // BN,),
    in_specs=[
        pl.BlockSpec((BN, D), lambda i: (i, 0)),
        pl.BlockSpec((D, D), lambda i: (0, 0)),
        pl.BlockSpec((D, D), lambda i: (0, 0)),
        pl.BlockSpec((1, D), lambda i: (0, 0)),
    ],
    out_specs=[
        pl.BlockSpec((BN, D), lambda i: (i, 0)),
        pl.BlockSpec((BN, D), lambda i: (i, 0)),
    ],
    out_shape=[
        jax.ShapeDtypeStruct((N, D), jnp.float32),
        jax.ShapeDtypeStruct((N, D), jnp.float32),
    ],
)


def _mid_body(h1_ref, w2_ref, b2_ref, v_ref, out_ref):
    h2 = jnp.maximum(jnp.dot(h1_ref[...], w2_ref[...],
                             preferred_element_type=jnp.float32)
                     + b2_ref[...], 0.0)
    out_ref[...] = h2 + v_ref[...]


def _make_mid(ecall):
    return pl.pallas_call(
        _mid_body,
        grid=(ecall // BM,),
        in_specs=[
            pl.BlockSpec((BM, D), lambda i: (i, 0)),
            pl.BlockSpec((D, D), lambda i: (0, 0)),
            pl.BlockSpec((1, D), lambda i: (0, 0)),
            pl.BlockSpec((1, D), lambda i: (0, 0)),
        ],
        out_specs=pl.BlockSpec((BM, D), lambda i: (i, 0)),
        out_shape=jax.ShapeDtypeStruct((ecall, D), jnp.float32),
    )


_MID_CALLS = [_make_mid(_ec) for _ec in SPLITS]


def _pinbox_body(a_ref, w3_ref, o_ref):
    acc = a_ref[0] + a_ref[1]
    o_ref[...] = jnp.dot(acc, w3_ref[...], precision=HI)


_pinbox = pl.pallas_call(
    _pinbox_body,
    grid=(N // BN,),
    in_specs=[
        pl.BlockSpec((2, BN, D), lambda i: (0, i, 0)),
        pl.BlockSpec((D, D), lambda i: (0, 0)),
    ],
    out_specs=pl.BlockSpec((BN, D), lambda i: (i, 0)),
    out_shape=jax.ShapeDtypeStruct((N, D), jnp.float32),
)


def _post_body(i0_ref, i1_ref, x_ref,
               nw1r_ref, nw1s_ref, nb1_ref, nw2_ref, nb2_ref,
               nw3_ref, nb3_ref, o_ref):
    inbox = i0_ref[...] + i1_ref[...]
    x = x_ref[...]
    u1 = jnp.maximum(jnp.dot(x, nw1r_ref[...], precision=HI)
                     + jnp.dot(inbox, nw1s_ref[...], precision=HI)
                     + nb1_ref[...], 0.0)
    u2 = jnp.maximum(jnp.dot(u1, nw2_ref[...], precision=HI)
                     + nb2_ref[...], 0.0)
    o_ref[...] = x + jnp.dot(u2, nw3_ref[...], precision=HI) + nb3_ref[...]


_W = pl.BlockSpec((D, D), lambda i: (0, 0))
_B = pl.BlockSpec((1, D), lambda i: (0, 0))
_A = pl.BlockSpec((BN, D), lambda i: (i, 0))
_post = pl.pallas_call(
    _post_body,
    grid=(N // BN,),
    in_specs=[_A, _A, _A, _W, _W, _B, _W, _B, _W, _B],
    out_specs=pl.BlockSpec((BN, D), lambda i: (i, 0)),
    out_shape=jax.ShapeDtypeStruct((N, D), jnp.float32),
)


def kernel(nodes, senders, receivers, mW1, mb1, mW2, mb2, mW3, mb3,
           nW1, nb1, nW2, nb2, nW3, nb3):
    x = nodes[0]
    v = jnp.linalg.solve(mW3.T, mb3).reshape(1, D)
    p, q = _pre(x, mW1[:D], mW1[D:], mb1.reshape(1, D))
    mb2r = mb2.reshape(1, D)
    inboxes = []
    for edge_call, mid_call, scatter_call in zip(
            _EDGE_CALLS, _MID_CALLS, _SCATTER_CALLS):
        h1 = edge_call(p, q, receivers, senders)
        h2 = mid_call(h1, mW2, mb2r, v)
        a2 = scatter_call(h2, receivers)
        inboxes.append(_pinbox(a2, mW3))
    out = _post(*inboxes, x,
                nW1[:D], nW1[D:], nb1.reshape(1, D),
                nW2, nb2.reshape(1, D), nW3, nb3.reshape(1, D))
    return out[None]


# speedup vs baseline: 1.0629x; 1.0629x over previous
import functools

import jax
import jax.numpy as jnp
from jax import lax
from jax.experimental import pallas as pl
from jax.experimental.pallas import tpu as pltpu
from jax.experimental.pallas import tpu_sc as plsc

N = 10000
E = 320000
D = 128
NC, NS = 2, 16
NW = NC * NS
CH = 80
NPAD = 10240
RPT = NPAD // NS
ZR = 64
BM = 2000
BN = 2000
SPLITS = (192000, 128000)

_MESH = plsc.VectorSubcoreMesh(core_axis_name="c", subcore_axis_name="s")


def _make_sc_edge(call_base, ecall):
    epw = ecall // NW
    nchunk = epw // CH

    @functools.partial(
        pl.kernel,
        mesh=_MESH,
        out_type=jax.ShapeDtypeStruct((ecall, D), jnp.float32),
        scratch_types=(
            [pltpu.VMEM((epw,), jnp.int32)] * 2
            + [pltpu.VMEM((CH, D), jnp.float32)] * 9
            + [pltpu.SemaphoreType.DMA] * 9
        ),
    )
    def sc_edge(p_hbm, q_hbm, recv_hbm, send_hbm, h1_hbm,
                ridx, sidx, prow0, prow1, prow2, qrow0, qrow1, qrow2,
                h1v0, h1v1, h1v2,
                gp0, gp1, gp2, gq0, gq1, gq2, wr0, wr1, wr2):
        wid = lax.axis_index("s") * NC + lax.axis_index("c")
        gbase = call_base + wid * epw
        obase = wid * epw
        prow = (prow0, prow1, prow2)
        qrow = (qrow0, qrow1, qrow2)
        h1v = (h1v0, h1v1, h1v2)
        gp = (gp0, gp1, gp2)
        gq = (gq0, gq1, gq2)
        wr = (wr0, wr1, wr2)

        pltpu.sync_copy(recv_hbm.at[pl.ds(gbase, epw)], ridx)
        pltpu.sync_copy(send_hbm.at[pl.ds(gbase, epw)], sidx)

        def issue_gather(cc, b):
            off = cc * CH
            pltpu.async_copy(p_hbm.at[ridx.at[pl.ds(off, CH)]], prow[b], gp[b])
            pltpu.async_copy(q_hbm.at[sidx.at[pl.ds(off, CH)]], qrow[b], gq[b])

        def wait_gather(b):
            pltpu.make_async_copy(p_hbm.at[pl.ds(0, CH)], prow[b], gp[b]).wait()
            pltpu.make_async_copy(q_hbm.at[pl.ds(0, CH)], qrow[b], gq[b]).wait()

        def issue_wb(cc, b):
            pltpu.async_copy(h1v[b], h1_hbm.at[pl.ds(obase + cc * CH, CH)], wr[b])

        def wait_wb(b):
            pltpu.make_async_copy(h1v[b], h1_hbm.at[pl.ds(obase, CH)], wr[b]).wait()

        issue_gather(0, 0)
        issue_gather(1, 1)

        @pl.loop(0, nchunk + 2, step=3)
        def _(k):
            for j in range(3):
                b = j
                cc = k + j

                @pl.when(cc < nchunk)
                def _():
                    @pl.when(cc + 2 < nchunk)
                    def _():
                        issue_gather(cc + 2, (b + 2) % 3)

                    wait_gather(b)

                    @pl.when(cc >= 3)
                    def _():
                        wait_wb(b)

                    @pl.loop(0, CH, step=2)
                    def _(r):
                        for rr in range(2):
                            for c in range(0, D, 16):
                                a = prow[b].at[pl.ds(r + rr, 1), pl.ds(c, 16)][...]
                                q = qrow[b].at[pl.ds(r + rr, 1), pl.ds(c, 16)][...]
                                h1v[b].at[pl.ds(r + rr, 1), pl.ds(c, 16)][...] = (
                                    jnp.maximum(a + q, 0.0))

                    issue_wb(cc, b)

        wait_wb((nchunk - 3) % 3)
        wait_wb((nchunk - 2) % 3)
        wait_wb((nchunk - 1) % 3)

    return sc_edge


def _make_sc_scatter(call_base, ecall):
    epw = ecall // NW
    nchunk = epw // CH

    @functools.partial(
        pl.kernel,
        mesh=_MESH,
        out_type=jax.ShapeDtypeStruct((NC, NPAD, D), jnp.float32),
        scratch_types=(
            [pltpu.VMEM_SHARED((NPAD, D), jnp.float32),
             pltpu.VMEM((epw,), jnp.int32)]
            + [pltpu.VMEM((CH, D), jnp.float32)] * 3
            + [pltpu.VMEM((ZR, D), jnp.float32)]
            + [pltpu.SemaphoreType.DMA] * 6
        ),
    )
    def sc_scatter(h2_hbm, recv_hbm, out_hbm, acc_sh, cidx, upd0, upd1, upd2,
                   stage, ld0, ld1, ld2, sc0, sc1, sc2):
        cid = lax.axis_index("c")
        sid = lax.axis_index("s")
        upd = (upd0, upd1, upd2)
        ld = (ld0, ld1, ld2)
        sc = (sc0, sc1, sc2)

        wid = sid * NC + cid
        gbase = call_base + wid * epw
        hbase = wid * epw

        @pl.loop(0, ZR)
        def _(r):
            for c in range(0, D, 16):
                stage.at[pl.ds(r, 1), pl.ds(c, 16)][...] = (
                    jnp.zeros((1, 16), jnp.float32))

        rbase = sid * RPT

        @pl.loop(0, RPT, step=ZR)
        def _(r):
            pltpu.sync_copy(stage, acc_sh.at[pl.ds(rbase + r, ZR)])

        pltpu.sync_copy(recv_hbm.at[pl.ds(gbase, epw)], cidx)
        plsc.subcore_barrier()

        def issue_load(cc, b):
            pltpu.async_copy(h2_hbm.at[pl.ds(hbase + cc * CH, CH)], upd[b], ld[b])

        def wait_load(b):
            pltpu.make_async_copy(h2_hbm.at[pl.ds(hbase, CH)], upd[b], ld[b]).wait()

        def issue_scatter(cc, b):
            off = cc * CH
            pltpu.async_copy(upd[b], acc_sh.at[cidx.at[pl.ds(off, CH)]], sc[b],
                             add=True)

        def wait_scatter(b):
            pltpu.make_async_copy(h2_hbm.at[pl.ds(hbase, CH)], upd[b], sc[b]).wait()

        issue_load(0, 0)

        @pl.loop(0, nchunk + 2, step=3)
        def _(k):
            for j in range(3):
                b = j
                cc = k + j

                @pl.when(cc < nchunk)
                def _():
                    wait_load(b)
                    issue_scatter(cc, b)

                    @pl.when(cc + 1 < nchunk)
                    def _():
                        @pl.when(cc >= 2)
                        def _():
                            wait_scatter((b + 1) % 3)

                        issue_load(cc + 1, (b + 1) % 3)

        wait_scatter((nchunk - 3) % 3)
        wait_scatter((nchunk - 2) % 3)
        wait_scatter((nchunk - 1) % 3)
        plsc.subcore_barrier()

        @pl.loop(0, RPT, step=ZR)
        def _(r):
            pltpu.sync_copy(acc_sh.at[pl.ds(rbase + r, ZR)], stage)
            pltpu.sync_copy(stage, out_hbm.at[cid, pl.ds(rbase + r, ZR)])

    return sc_scatter


_EDGE_CALLS = []
_SCATTER_CALLS = []
_base = 0
for _ec in SPLITS:
    _EDGE_CALLS.append(_make_sc_edge(_base, _ec))
    _SCATTER_CALLS.append(_make_sc_scatter(_base, _ec))
    _base += _ec


def _pre_body(x_ref, w1r_ref, w1s_ref, b1_ref, p_ref, q_ref):
    x = x_ref[...]
    p_ref[...] = jnp.dot(x, w1r_ref[...], preferred_element_type=jnp.float32) + b1_ref[...]
    q_ref[...] = jnp.dot(x, w1s_ref[...], preferred_element_type=jnp.float32)


_pre = pl.pallas_call(
    _pre_body,
    grid=(N // BN,),
    in_specs=[
        pl.BlockSpec((BN, D), lambda i: (i, 0)),
        pl.BlockSpec((D, D), lambda i: (0, 0)),
        pl.BlockSpec((D, D), lambda i: (0, 0)),
        pl.BlockSpec((1, D), lambda i: (0, 0)),
    ],
    out_specs=[
        pl.BlockSpec((BN, D), lambda i: (i, 0)),
        pl.BlockSpec((BN, D), lambda i: (i, 0)),
    ],
    out_shape=[
        jax.ShapeDtypeStruct((N, D), jnp.float32),
        jax.ShapeDtypeStruct((N, D), jnp.float32),
    ],
)


def _mid_body(h1_ref, w2_ref, b2_ref, v_ref, out_ref):
    h2 = jnp.maximum(jnp.dot(h1_ref[...], w2_ref[...],
                             preferred_element_type=jnp.float32)
                     + b2_ref[...], 0.0)
    out_ref[...] = h2 + v_ref[...]


def _make_mid(ecall):
    return pl.pallas_call(
        _mid_body,
        grid=(ecall // BM,),
        in_specs=[
            pl.BlockSpec((BM, D), lambda i: (i, 0)),
            pl.BlockSpec((D, D), lambda i: (0, 0)),
            pl.BlockSpec((1, D), lambda i: (0, 0)),
            pl.BlockSpec((1, D), lambda i: (0, 0)),
        ],
        out_specs=pl.BlockSpec((BM, D), lambda i: (i, 0)),
        out_shape=jax.ShapeDtypeStruct((ecall, D), jnp.float32),
    )


_MID_CALLS = [_make_mid(_ec) for _ec in SPLITS]


def _pinbox_body(a_ref, w3_ref, o_ref):
    acc = a_ref[0] + a_ref[1]
    o_ref[...] = jnp.dot(acc, w3_ref[...], preferred_element_type=jnp.float32)


_pinbox = pl.pallas_call(
    _pinbox_body,
    grid=(N // BN,),
    in_specs=[
        pl.BlockSpec((2, BN, D), lambda i: (0, i, 0)),
        pl.BlockSpec((D, D), lambda i: (0, 0)),
    ],
    out_specs=pl.BlockSpec((BN, D), lambda i: (i, 0)),
    out_shape=jax.ShapeDtypeStruct((N, D), jnp.float32),
)


def _post_body(i0_ref, i1_ref, x_ref,
               nw1r_ref, nw1s_ref, nb1_ref, nw2_ref, nb2_ref,
               nw3_ref, nb3_ref, o_ref):
    inbox = i0_ref[...] + i1_ref[...]
    x = x_ref[...]
    u1 = jnp.maximum(jnp.dot(x, nw1r_ref[...], preferred_element_type=jnp.float32)
                     + jnp.dot(inbox, nw1s_ref[...], preferred_element_type=jnp.float32)
                     + nb1_ref[...], 0.0)
    u2 = jnp.maximum(jnp.dot(u1, nw2_ref[...], preferred_element_type=jnp.float32)
                     + nb2_ref[...], 0.0)
    o_ref[...] = x + jnp.dot(u2, nw3_ref[...], preferred_element_type=jnp.float32) + nb3_ref[...]


_W = pl.BlockSpec((D, D), lambda i: (0, 0))
_B = pl.BlockSpec((1, D), lambda i: (0, 0))
_A = pl.BlockSpec((BN, D), lambda i: (i, 0))
_post = pl.pallas_call(
    _post_body,
    grid=(N // BN,),
    in_specs=[_A, _A, _A, _W, _W, _B, _W, _B, _W, _B],
    out_specs=pl.BlockSpec((BN, D), lambda i: (i, 0)),
    out_shape=jax.ShapeDtypeStruct((N, D), jnp.float32),
)


def kernel(nodes, senders, receivers, mW1, mb1, mW2, mb2, mW3, mb3,
           nW1, nb1, nW2, nb2, nW3, nb3):
    x = nodes[0]
    v = jnp.linalg.solve(mW3.T, mb3).reshape(1, D)
    p, q = _pre(x, mW1[:D], mW1[D:], mb1.reshape(1, D))
    mb2r = mb2.reshape(1, D)
    inboxes = []
    for edge_call, mid_call, scatter_call in zip(
            _EDGE_CALLS, _MID_CALLS, _SCATTER_CALLS):
        h1 = edge_call(p, q, receivers, senders)
        h2 = mid_call(h1, mW2, mb2r, v)
        a2 = scatter_call(h2, receivers)
        inboxes.append(_pinbox(a2, mW3))
    out = _post(*inboxes, x,
                nW1[:D], nW1[D:], nb1.reshape(1, D),
                nW2, nb2.reshape(1, D), nW3, nb3.reshape(1, D))
    return out[None]
